# Initial kernel scaffold; baseline (speedup 1.0000x reference)
import jax
import jax.numpy as jnp
from jax.experimental import pallas as pl


def kernel(x, edge_index, W_root, W_rel, b_enc, W_dec, b_dec):
    raise NotImplementedError("write your pallas kernel here")


# speedup vs baseline: 7.6812x; 7.6812x over previous
import jax
import jax.numpy as jnp
from jax import lax
from jax.experimental import pallas as pl
from jax.experimental.pallas import tpu as pltpu
from jax.experimental.pallas import tpu_sc as plsc

N = 10000
E = 320000
IN_CH = 128
HID = 64

NC = 2
NS = 16
NW = NC * NS
CHUNK = 128
CPT = -(-E // (NW * CHUNK))
E_PAD = NW * CPT * CHUNK
NPAD = 10240
NP2 = NPAD // 2
SLICE_ENC = NPAD // NS
SLICE2 = NP2 // NS


def _enc_body(x_ref, w_ref, out_ref):
    out_ref[...] = jnp.dot(x_ref[...], w_ref[0],
                           preferred_element_type=jnp.float32)


def _sc_body(gidx_ref, sidx_ref, ywz_ref, zero_ref, out_ref,
             gidx, sidx, rows, stage, agg, sem):
    c = lax.axis_index("c")
    s = lax.axis_index("s")
    w = s * NC + c
    pltpu.sync_copy(zero_ref, stage)
    pltpu.sync_copy(stage, agg.at[pl.ds(s * SLICE2, SLICE2)])
    pltpu.sync_copy(gidx_ref.at[w], gidx)
    pltpu.sync_copy(sidx_ref.at[w], sidx)
    plsc.subcore_barrier()

    def step(j, carry):
        pltpu.async_copy(ywz_ref.at[gidx.at[j]], rows, sem).wait()
        pltpu.sync_copy(rows, agg.at[sidx.at[j]], add=True)
        return carry

    lax.fori_loop(0, CPT, step, 0)
    plsc.subcore_barrier()
    pltpu.sync_copy(agg.at[pl.ds(s * SLICE2, SLICE2)], stage)
    pltpu.sync_copy(stage, out_ref.at[c, pl.ds(s * SLICE2, SLICE2)])


_sc_scatter = pl.kernel(
    _sc_body,
    out_type=jax.ShapeDtypeStruct((NC, NP2, IN_CH), jnp.float32),
    mesh=plsc.VectorSubcoreMesh(core_axis_name="c", subcore_axis_name="s"),
    scratch_types=[
        pltpu.VMEM((CPT, CHUNK), jnp.int32),
        pltpu.VMEM((CPT, CHUNK), jnp.int32),
        pltpu.VMEM((CHUNK, IN_CH), jnp.float32),
        pltpu.VMEM((SLICE2, IN_CH), jnp.float32),
        pltpu.VMEM_SHARED((NP2, IN_CH), jnp.float32),
        pltpu.SemaphoreType.DMA,
    ],
)


def _dec_body(x_ref, p0_ref, p1_ref, wroot_ref, benc_ref,
              wdec_ref, bdec_ref, out_ref):
    h = (
        jnp.dot(x_ref[...], wroot_ref[...], preferred_element_type=jnp.float32)
        + p0_ref[...] + p1_ref[...] + benc_ref[...]
    )
    h = jnp.maximum(h, 0.0)
    out_ref[...] = (
        jnp.dot(h, wdec_ref[...], preferred_element_type=jnp.float32)
        + bdec_ref[...]
    )


_ROWS_BLK = 1000


@jax.jit
def _forward(x, edge_index, W_root, W_rel, b_enc, W_dec, b_dec):
    src = edge_index[0].astype(jnp.int32)
    dst = edge_index[1].astype(jnp.int32)
    pad = E_PAD - E
    pad_src = (jnp.arange(pad, dtype=jnp.int32) * 89) % N
    pad_dst = N + (jnp.arange(pad, dtype=jnp.int32) % (NPAD - N))
    src_p = jnp.concatenate([src, pad_src])
    dst_p = jnp.concatenate([dst, pad_dst])
    gidx = (src_p + NPAD * (dst_p & 1)).reshape(NW, CPT, CHUNK)
    sidx = (dst_p >> 1).reshape(NW, CPT, CHUNK)

    w_stack = jnp.stack([
        jnp.concatenate([W_rel, jnp.zeros((IN_CH, HID), jnp.float32)], axis=1),
        jnp.concatenate([jnp.zeros((IN_CH, HID), jnp.float32), W_rel], axis=1),
    ])
    ywz = pl.pallas_call(
        _enc_body,
        grid=(2 * NS,),
        in_specs=[
            pl.BlockSpec((SLICE_ENC, IN_CH), lambda i: (i % NS, 0)),
            pl.BlockSpec((1, IN_CH, IN_CH), lambda i: (i // NS, 0, 0)),
        ],
        out_specs=pl.BlockSpec((SLICE_ENC, IN_CH), lambda i: (i, 0)),
        out_shape=jax.ShapeDtypeStruct((2 * NPAD, IN_CH), jnp.float32),
    )(x, w_stack)

    zero = jnp.zeros((SLICE2, IN_CH), jnp.float32)
    parts = _sc_scatter(gidx, sidx, ywz, zero)
    p0 = parts[0].reshape(NPAD, HID)[:N]
    p1 = parts[1].reshape(NPAD, HID)[:N]

    out = pl.pallas_call(
        _dec_body,
        grid=(N // _ROWS_BLK,),
        in_specs=[
            pl.BlockSpec((_ROWS_BLK, IN_CH), lambda i: (i, 0)),
            pl.BlockSpec((_ROWS_BLK, HID), lambda i: (i, 0)),
            pl.BlockSpec((_ROWS_BLK, HID), lambda i: (i, 0)),
            pl.BlockSpec((IN_CH, HID), lambda i: (0, 0)),
            pl.BlockSpec((1, HID), lambda i: (0, 0)),
            pl.BlockSpec((HID, IN_CH), lambda i: (0, 0)),
            pl.BlockSpec((1, IN_CH), lambda i: (0, 0)),
        ],
        out_specs=pl.BlockSpec((_ROWS_BLK, IN_CH), lambda i: (i, 0)),
        out_shape=jax.ShapeDtypeStruct((N, IN_CH), jnp.float32),
    )(x, p0, p1, W_root, b_enc.reshape(1, HID), W_dec,
      b_dec.reshape(1, IN_CH))
    return out


def kernel(x, edge_index, W_root, W_rel, b_enc, W_dec, b_dec):
    return _forward(x, edge_index, W_root, W_rel, b_enc, W_dec, b_dec)


# speedup vs baseline: 7.8961x; 1.0280x over previous
import jax
import jax.numpy as jnp
from jax import lax
from jax.experimental import pallas as pl
from jax.experimental.pallas import tpu as pltpu
from jax.experimental.pallas import tpu_sc as plsc

N = 10000
E = 320000
IN_CH = 128
HID = 64

NC = 2
NS = 16
NW = NC * NS
CHUNK = 128
CPT = 2 * (-(-E // (NW * CHUNK * 2)))
E_PAD = NW * CPT * CHUNK
NPAD = 10240
NP2 = NPAD // 2
SLICE_ENC = NPAD // NS
SLICE2 = NP2 // NS


def _enc_body(x_ref, w_ref, out_ref):
    out_ref[...] = jnp.dot(x_ref[...].astype(jnp.bfloat16),
                           w_ref[0].astype(jnp.bfloat16),
                           preferred_element_type=jnp.float32)


def _sc_body(gidx_ref, sidx_ref, ywz_ref, zero_ref, out_ref,
             gidx, sidx, rows, stage, agg, sem0):
    c = lax.axis_index("c")
    s = lax.axis_index("s")
    w = s * NC + c
    zv = jnp.zeros((16,), jnp.float32)

    def zrow(i, carry):
        for jj in range(8):
            stage[i, pl.ds(jj * 16, 16)] = zv
        return carry

    lax.fori_loop(0, SLICE2, zrow, 0)
    pltpu.sync_copy(stage, agg.at[pl.ds(s * SLICE2, SLICE2)])
    pltpu.sync_copy(gidx_ref.at[w], gidx)
    pltpu.sync_copy(sidx_ref.at[w], sidx)
    plsc.subcore_barrier()

    def step(j, carry):
        pltpu.async_copy(ywz_ref.at[gidx.at[j]], rows, sem0).wait()
        pltpu.sync_copy(rows, agg.at[sidx.at[j]], add=True)
        return carry

    lax.fori_loop(0, CPT, step, 0)
    plsc.subcore_barrier()
    pltpu.sync_copy(agg.at[pl.ds(s * SLICE2, SLICE2)], stage)
    pltpu.sync_copy(stage, out_ref.at[c, pl.ds(s * SLICE2, SLICE2)])


_sc_scatter = pl.kernel(
    _sc_body,
    out_type=jax.ShapeDtypeStruct((NC, NP2, IN_CH), jnp.float32),
    mesh=plsc.VectorSubcoreMesh(core_axis_name="c", subcore_axis_name="s"),
    scratch_types=[
        pltpu.VMEM((CPT, CHUNK), jnp.int32),
        pltpu.VMEM((CPT, CHUNK), jnp.int32),
        pltpu.VMEM((CHUNK, IN_CH), jnp.float32),
        pltpu.VMEM((SLICE2, IN_CH), jnp.float32),
        pltpu.VMEM_SHARED((NP2, IN_CH), jnp.float32),
        pltpu.SemaphoreType.DMA,
    ],
)


def _dec_body(x_ref, p0_ref, p1_ref, wroot_ref, benc_ref,
              wdec_ref, bdec_ref, out_ref):
    agg = p0_ref[...] + p1_ref[...]
    h = (
        jnp.dot(x_ref[...].astype(jnp.bfloat16),
                wroot_ref[...].astype(jnp.bfloat16),
                preferred_element_type=jnp.float32)
        + agg + benc_ref[...]
    )
    h = jnp.maximum(h, 0.0)
    out_ref[...] = (
        jnp.dot(h.astype(jnp.bfloat16), wdec_ref[...].astype(jnp.bfloat16),
                preferred_element_type=jnp.float32)
        + bdec_ref[...]
    )


_ROWS_BLK = 1280


@jax.jit
def _forward(x, edge_index, W_root, W_rel, b_enc, W_dec, b_dec):
    src = edge_index[0].astype(jnp.int32)
    dst = edge_index[1].astype(jnp.int32)
    pad = E_PAD - E
    pad_src = (jnp.arange(pad, dtype=jnp.int32) * 89) % N
    pad_dst = N + (jnp.arange(pad, dtype=jnp.int32) % (NPAD - N))
    src_p = jnp.concatenate([src, pad_src])
    dst_p = jnp.concatenate([dst, pad_dst])
    gidx = (src_p + NPAD * (dst_p & 1)).reshape(NW, CPT, CHUNK)
    sidx = (dst_p >> 1).reshape(NW, CPT, CHUNK)

    w_stack = jnp.stack([
        jnp.concatenate([W_rel, jnp.zeros((IN_CH, HID), jnp.float32)], axis=1),
        jnp.concatenate([jnp.zeros((IN_CH, HID), jnp.float32), W_rel], axis=1),
    ])
    ywz = pl.pallas_call(
        _enc_body,
        grid=(2 * NS,),
        in_specs=[
            pl.BlockSpec((SLICE_ENC, IN_CH), lambda i: (i % NS, 0)),
            pl.BlockSpec((1, IN_CH, IN_CH), lambda i: (i // NS, 0, 0)),
        ],
        out_specs=pl.BlockSpec((SLICE_ENC, IN_CH), lambda i: (i, 0)),
        out_shape=jax.ShapeDtypeStruct((2 * NPAD, IN_CH), jnp.float32),
    )(x, w_stack)

    zero = jnp.zeros((SLICE2, IN_CH), jnp.float32)
    parts = _sc_scatter(gidx, sidx, ywz, zero)
    p0 = parts[0].reshape(NPAD, HID)
    p1 = parts[1].reshape(NPAD, HID)

    out = pl.pallas_call(
        _dec_body,
        grid=(NPAD // _ROWS_BLK,),
        in_specs=[
            pl.BlockSpec((_ROWS_BLK, IN_CH), lambda i: (i, 0)),
            pl.BlockSpec((_ROWS_BLK, HID), lambda i: (i, 0)),
            pl.BlockSpec((_ROWS_BLK, HID), lambda i: (i, 0)),
            pl.BlockSpec((IN_CH, HID), lambda i: (0, 0)),
            pl.BlockSpec((1, HID), lambda i: (0, 0)),
            pl.BlockSpec((HID, IN_CH), lambda i: (0, 0)),
            pl.BlockSpec((1, IN_CH), lambda i: (0, 0)),
        ],
        out_specs=pl.BlockSpec((_ROWS_BLK, IN_CH), lambda i: (i, 0)),
        out_shape=jax.ShapeDtypeStruct((N, IN_CH), jnp.float32),
    )(x, p0, p1, W_root, b_enc.reshape(1, HID), W_dec,
      b_dec.reshape(1, IN_CH))
    return out


def kernel(x, edge_index, W_root, W_rel, b_enc, W_dec, b_dec):
    return _forward(x, edge_index, W_root, W_rel, b_enc, W_dec, b_dec)


# speedup vs baseline: 10.5381x; 1.3346x over previous
import jax
import jax.numpy as jnp
from jax import lax
from jax.experimental import pallas as pl
from jax.experimental.pallas import tpu as pltpu
from jax.experimental.pallas import tpu_sc as plsc

N = 10000
E = 320000
IN_CH = 128
HID = 64

NC = 2
NS = 16
NW = NC * NS
CHUNK = 128
CPT = -(-E // (NW * CHUNK))
E_PAD = NW * CPT * CHUNK
NPAD = 10240
SLICE = NPAD // NS


def _enc_body(x_ref, w_ref, out_ref):
    out_ref[...] = jnp.dot(x_ref[...].astype(jnp.bfloat16),
                           w_ref[...].astype(jnp.bfloat16),
                           preferred_element_type=jnp.float32)


def _sc_body(gidx_ref, sidx_ref, y_ref, out_ref,
             gidx, sidx, rows, stage, agg, sem0):
    c = lax.axis_index("c")
    s = lax.axis_index("s")
    w = s * NC + c
    zv = jnp.zeros((16,), jnp.float32)

    def zrow(i, carry):
        for jj in range(HID // 16):
            stage[i, pl.ds(jj * 16, 16)] = zv
        return carry

    lax.fori_loop(0, SLICE, zrow, 0)
    pltpu.sync_copy(stage, agg.at[pl.ds(s * SLICE, SLICE)])
    pltpu.sync_copy(gidx_ref.at[w], gidx)
    pltpu.sync_copy(sidx_ref.at[w], sidx)
    plsc.subcore_barrier()

    def step(j, carry):
        pltpu.async_copy(y_ref.at[gidx.at[j]], rows, sem0).wait()
        pltpu.sync_copy(rows, agg.at[sidx.at[j]], add=True)
        return carry

    lax.fori_loop(0, CPT, step, 0)
    plsc.subcore_barrier()
    pltpu.sync_copy(agg.at[pl.ds(s * SLICE, SLICE)], stage)
    pltpu.sync_copy(stage, out_ref.at[c, pl.ds(s * SLICE, SLICE)])


_sc_scatter = pl.kernel(
    _sc_body,
    out_type=jax.ShapeDtypeStruct((NC, NPAD, HID), jnp.float32),
    mesh=plsc.VectorSubcoreMesh(core_axis_name="c", subcore_axis_name="s"),
    compiler_params=pltpu.CompilerParams(use_tc_tiling_on_sc=False),
    scratch_types=[
        pltpu.VMEM((CPT, CHUNK), jnp.int32),
        pltpu.VMEM((CPT, CHUNK), jnp.int32),
        pltpu.VMEM((CHUNK, HID), jnp.float32),
        pltpu.VMEM((SLICE, HID), jnp.float32),
        pltpu.VMEM_SHARED((NPAD, HID), jnp.float32),
        pltpu.SemaphoreType.DMA,
    ],
)


def _dec_body(x_ref, p0_ref, p1_ref, wroot_ref, benc_ref,
              wdec_ref, bdec_ref, out_ref):
    agg = p0_ref[0] + p1_ref[0]
    h = (
        jnp.dot(x_ref[...].astype(jnp.bfloat16),
                wroot_ref[...].astype(jnp.bfloat16),
                preferred_element_type=jnp.float32)
        + agg + benc_ref[...]
    )
    h = jnp.maximum(h, 0.0)
    out_ref[...] = (
        jnp.dot(h.astype(jnp.bfloat16), wdec_ref[...].astype(jnp.bfloat16),
                preferred_element_type=jnp.float32)
        + bdec_ref[...]
    )


_ROWS_BLK = 1280


@jax.jit
def _forward(x, edge_index, W_root, W_rel, b_enc, W_dec, b_dec):
    src = edge_index[0].astype(jnp.int32)
    dst = edge_index[1].astype(jnp.int32)
    pad = E_PAD - E
    pad_src = (jnp.arange(pad, dtype=jnp.int32) * 89) % N
    pad_dst = N + (jnp.arange(pad, dtype=jnp.int32) % (NPAD - N))
    gidx = jnp.concatenate([src, pad_src]).reshape(NW, CPT, CHUNK)
    sidx = jnp.concatenate([dst, pad_dst]).reshape(NW, CPT, CHUNK)

    y = pl.pallas_call(
        _enc_body,
        grid=(NS,),
        in_specs=[
            pl.BlockSpec((SLICE, IN_CH), lambda i: (i, 0)),
            pl.BlockSpec((IN_CH, HID), lambda i: (0, 0)),
        ],
        out_specs=pl.BlockSpec((SLICE, HID), lambda i: (i, 0)),
        out_shape=jax.ShapeDtypeStruct((NPAD, HID), jnp.float32),
    )(x, W_rel)

    parts = _sc_scatter(gidx, sidx, y)

    out = pl.pallas_call(
        _dec_body,
        grid=(NPAD // _ROWS_BLK,),
        in_specs=[
            pl.BlockSpec((_ROWS_BLK, IN_CH), lambda i: (i, 0)),
            pl.BlockSpec((1, _ROWS_BLK, HID), lambda i: (0, i, 0)),
            pl.BlockSpec((1, _ROWS_BLK, HID), lambda i: (1, i, 0)),
            pl.BlockSpec((IN_CH, HID), lambda i: (0, 0)),
            pl.BlockSpec((1, HID), lambda i: (0, 0)),
            pl.BlockSpec((HID, IN_CH), lambda i: (0, 0)),
            pl.BlockSpec((1, IN_CH), lambda i: (0, 0)),
        ],
        out_specs=pl.BlockSpec((_ROWS_BLK, IN_CH), lambda i: (i, 0)),
        out_shape=jax.ShapeDtypeStruct((N, IN_CH), jnp.float32),
    )(x, parts, parts, W_root, b_enc.reshape(1, HID), W_dec,
      b_dec.reshape(1, IN_CH))
    return out


def kernel(x, edge_index, W_root, W_rel, b_enc, W_dec, b_dec):
    return _forward(x, edge_index, W_root, W_rel, b_enc, W_dec, b_dec)


# speedup vs baseline: 14.1930x; 1.3468x over previous
import jax
import jax.numpy as jnp
from jax import lax
from jax.experimental import pallas as pl
from jax.experimental.pallas import tpu as pltpu
from jax.experimental.pallas import tpu_sc as plsc

N = 10000
E = 320000
IN_CH = 128
HID = 64

NC = 2
NS = 16
NW = NC * NS
CHUNK = 128
CPT = 2 * (-(-E // (NW * CHUNK * 2)))
E_PAD = NW * CPT * CHUNK
NPAD = 10240
SLICE = NPAD // NS


def _enc_body(x_ref, w_ref, out_ref):
    out_ref[...] = jnp.dot(x_ref[...].astype(jnp.bfloat16),
                           w_ref[...].astype(jnp.bfloat16),
                           preferred_element_type=jnp.float32)


def _sc_body(gidx_ref, sidx_ref, y_ref, out_ref,
             gidx, sidx, rows0, rows1, stage, agg, sem0, sem1):
    c = lax.axis_index("c")
    s = lax.axis_index("s")
    w = s * NC + c
    zv = jnp.zeros((16,), jnp.float32)

    def zrow(i, carry):
        for jj in range(HID // 16):
            stage[i, pl.ds(jj * 16, 16)] = zv
        return carry

    lax.fori_loop(0, SLICE, zrow, 0)
    pltpu.sync_copy(stage, agg.at[pl.ds(s * SLICE, SLICE)])
    pltpu.sync_copy(gidx_ref.at[w], gidx)
    pltpu.sync_copy(sidx_ref.at[w], sidx)
    plsc.subcore_barrier()

    pltpu.async_copy(y_ref.at[gidx.at[0]], rows0, sem0)

    def step(jj, carry):
        j0 = 2 * jj
        j1 = j0 + 1
        jn = jnp.minimum(j0 + 2, CPT - 2)
        pltpu.async_copy(y_ref.at[gidx.at[j1]], rows1, sem1)
        pltpu.make_async_copy(y_ref.at[gidx.at[j0]], rows0, sem0).wait()
        pltpu.sync_copy(rows0, agg.at[sidx.at[j0]], add=True)
        pltpu.async_copy(y_ref.at[gidx.at[jn]], rows0, sem0)
        pltpu.make_async_copy(y_ref.at[gidx.at[j1]], rows1, sem1).wait()
        pltpu.sync_copy(rows1, agg.at[sidx.at[j1]], add=True)
        return carry

    lax.fori_loop(0, CPT // 2, step, 0)
    pltpu.make_async_copy(y_ref.at[gidx.at[0]], rows0, sem0).wait()
    plsc.subcore_barrier()
    pltpu.sync_copy(agg.at[pl.ds(s * SLICE, SLICE)], stage)
    pltpu.sync_copy(stage, out_ref.at[c, pl.ds(s * SLICE, SLICE)])


_sc_scatter = pl.kernel(
    _sc_body,
    out_type=jax.ShapeDtypeStruct((NC, NPAD, HID), jnp.float32),
    mesh=plsc.VectorSubcoreMesh(core_axis_name="c", subcore_axis_name="s"),
    compiler_params=pltpu.CompilerParams(use_tc_tiling_on_sc=False),
    scratch_types=[
        pltpu.VMEM((CPT, CHUNK), jnp.int32),
        pltpu.VMEM((CPT, CHUNK), jnp.int32),
        pltpu.VMEM((CHUNK, HID), jnp.float32),
        pltpu.VMEM((CHUNK, HID), jnp.float32),
        pltpu.VMEM((SLICE, HID), jnp.float32),
        pltpu.VMEM_SHARED((NPAD, HID), jnp.float32),
        pltpu.SemaphoreType.DMA,
        pltpu.SemaphoreType.DMA,
    ],
)


def _dec_body(x_ref, p0_ref, p1_ref, wroot_ref, benc_ref,
              wdec_ref, bdec_ref, out_ref):
    agg = p0_ref[0] + p1_ref[0]
    h = (
        jnp.dot(x_ref[...].astype(jnp.bfloat16),
                wroot_ref[...].astype(jnp.bfloat16),
                preferred_element_type=jnp.float32)
        + agg + benc_ref[...]
    )
    h = jnp.maximum(h, 0.0)
    out_ref[...] = (
        jnp.dot(h.astype(jnp.bfloat16), wdec_ref[...].astype(jnp.bfloat16),
                preferred_element_type=jnp.float32)
        + bdec_ref[...]
    )


_ROWS_BLK = 1280


@jax.jit
def _forward(x, edge_index, W_root, W_rel, b_enc, W_dec, b_dec):
    src = edge_index[0].astype(jnp.int32)
    dst = edge_index[1].astype(jnp.int32)
    pad = E_PAD - E
    pad_src = (jnp.arange(pad, dtype=jnp.int32) * 89) % N
    pad_dst = N + (jnp.arange(pad, dtype=jnp.int32) % (NPAD - N))
    gidx = jnp.concatenate([src, pad_src]).reshape(NW, CPT, CHUNK)
    sidx = jnp.concatenate([dst, pad_dst]).reshape(NW, CPT, CHUNK)

    y = pl.pallas_call(
        _enc_body,
        grid=(NS,),
        in_specs=[
            pl.BlockSpec((SLICE, IN_CH), lambda i: (i, 0)),
            pl.BlockSpec((IN_CH, HID), lambda i: (0, 0)),
        ],
        out_specs=pl.BlockSpec((SLICE, HID), lambda i: (i, 0)),
        out_shape=jax.ShapeDtypeStruct((NPAD, HID), jnp.float32),
    )(x, W_rel)

    parts = _sc_scatter(gidx, sidx, y)

    out = pl.pallas_call(
        _dec_body,
        grid=(NPAD // _ROWS_BLK,),
        in_specs=[
            pl.BlockSpec((_ROWS_BLK, IN_CH), lambda i: (i, 0)),
            pl.BlockSpec((1, _ROWS_BLK, HID), lambda i: (0, i, 0)),
            pl.BlockSpec((1, _ROWS_BLK, HID), lambda i: (1, i, 0)),
            pl.BlockSpec((IN_CH, HID), lambda i: (0, 0)),
            pl.BlockSpec((1, HID), lambda i: (0, 0)),
            pl.BlockSpec((HID, IN_CH), lambda i: (0, 0)),
            pl.BlockSpec((1, IN_CH), lambda i: (0, 0)),
        ],
        out_specs=pl.BlockSpec((_ROWS_BLK, IN_CH), lambda i: (i, 0)),
        out_shape=jax.ShapeDtypeStruct((N, IN_CH), jnp.float32),
    )(x, parts, parts, W_root, b_enc.reshape(1, HID), W_dec,
      b_dec.reshape(1, IN_CH))
    return out


def kernel(x, edge_index, W_root, W_rel, b_enc, W_dec, b_dec):
    return _forward(x, edge_index, W_root, W_rel, b_enc, W_dec, b_dec)
